# Initial kernel scaffold; baseline (speedup 1.0000x reference)
import jax
import jax.numpy as jnp
from jax.experimental import pallas as pl


def kernel(distances, edge_features, residues, node_features, senders, receivers, node_graph_ids, emb_table, We1, be1, We2, be2, Wn1, bn1, Wn2, bn2, W_e, W_s, b_e, W_n, W_in, b_n, W_g, b_g, W_no, b_no):
    raise NotImplementedError("write your pallas kernel here")


# speedup vs baseline: 6.0634x; 6.0634x over previous
import functools

import numpy as np
import jax
import jax.numpy as jnp
from jax import lax
from jax.experimental import pallas as pl
from jax.experimental.pallas import tpu as pltpu
from jax.experimental.pallas import tpu_sc as plsc

N_NODES = 10000
N_PAD = 10240
N_EDGES = 320000
N_GRAPHS = 32
RBF_SIZE = 16
MIN_DIST = 0.0
MAX_DIST = 20.0

NC = 2
NS = 16
NW = NC * NS
EDGES_PER_W = N_EDGES // NW
BLK = 80
N_BLKS = EDGES_PER_W // BLK
N_PAIR = N_BLKS // 2
ACCW = 80
ROWS_PER_TILE = N_PAD // NS


def _edge_body(d_ref, ef_ref, we1_ref, be1_ref, we2_ref, be2_ref, we_ref, out_ref):
    d = d_ref[...]
    step = (MAX_DIST - MIN_DIST) / (RBF_SIZE - 1)
    centers = (lax.broadcasted_iota(jnp.int32, (1, RBF_SIZE), 1)
               .astype(jnp.float32) * step + MIN_DIST)
    rbf = jnp.exp(-(d - centers) ** 2)
    we1 = we1_ref[...]
    e4 = jnp.maximum(
        jnp.dot(rbf, we1[:RBF_SIZE], preferred_element_type=jnp.float32)
        + jnp.dot(ef_ref[...], we1[RBF_SIZE:], preferred_element_type=jnp.float32)
        + be1_ref[...][None, :], 0.0)
    e8 = jnp.maximum(
        jnp.dot(e4, we2_ref[...], preferred_element_type=jnp.float32)
        + be2_ref[...][None, :], 0.0)
    out_ref[...] = jnp.dot(e8, we_ref[...], preferred_element_type=jnp.float32)


def _edge_encoder(distances, edge_features, We1, be1, We2, be2, W_e):
    eb = 4000
    grid = N_EDGES // eb
    return pl.pallas_call(
        _edge_body,
        grid=(grid,),
        in_specs=[
            pl.BlockSpec((eb, 1), lambda i: (i, 0)),
            pl.BlockSpec((eb, 16), lambda i: (i, 0)),
            pl.BlockSpec((32, 4), lambda i: (0, 0)),
            pl.BlockSpec((4,), lambda i: (0,)),
            pl.BlockSpec((4, 8), lambda i: (0, 0)),
            pl.BlockSpec((8,), lambda i: (0,)),
            pl.BlockSpec((8, 64), lambda i: (0, 0)),
        ],
        out_specs=pl.BlockSpec((eb, 64), lambda i: (i, 0)),
        out_shape=jax.ShapeDtypeStruct((N_EDGES, 64), jnp.float32),
    )(distances.reshape(N_EDGES, 1), edge_features, We1, be1, We2, be2, W_e)


def _node_body(res_ref, nf_ref, emb_ref, wn1_ref, bn1_ref, wn2_ref, bn2_ref,
               ws_ref, be_ref, x16_ref, y64_ref):
    res = res_ref[...]
    onehot = (res == lax.broadcasted_iota(jnp.int32, (N_PAD, 22), 1)
              ).astype(jnp.float32)
    emb = jnp.dot(onehot, emb_ref[...], preferred_element_type=jnp.float32)
    wn1 = wn1_ref[...]
    x8 = jnp.maximum(
        jnp.dot(emb, wn1[:32], preferred_element_type=jnp.float32)
        + jnp.dot(nf_ref[...], wn1[32:], preferred_element_type=jnp.float32)
        + bn1_ref[...][None, :], 0.0)
    x16 = jnp.maximum(
        jnp.dot(x8, wn2_ref[...], preferred_element_type=jnp.float32)
        + bn2_ref[...][None, :], 0.0)
    x16_ref[...] = x16
    y64_ref[...] = (jnp.dot(x16, ws_ref[...], preferred_element_type=jnp.float32)
                    + be_ref[...][None, :])


def _node_encoder(residues, node_features, emb_table, Wn1, bn1, Wn2, bn2, W_s, b_e):
    res_p = jnp.zeros((N_PAD, 1), jnp.int32).at[:N_NODES, 0].set(
        residues.astype(jnp.int32))
    nf_p = jnp.zeros((N_PAD, 96), jnp.float32).at[:N_NODES].set(node_features)
    return pl.pallas_call(
        _node_body,
        out_shape=(jax.ShapeDtypeStruct((N_PAD, 16), jnp.float32),
                   jax.ShapeDtypeStruct((N_PAD, 64), jnp.float32)),
    )(res_p, nf_p, emb_table, Wn1, bn1, Wn2, bn2, W_s, b_e)


def _sc_body(c_hbm, snd_hbm, rcv_hbm, y64_hbm, zero_hbm, out_hbm,
             acc_sh,
             c_v0, snd_v0, rcv_v0, ys_v0, yr_v0, zf_v0, zb_v0, fs_v0, bs_v0,
             c_v1, snd_v1, rcv_v1, ys_v1, yr_v1, zf_v1, zb_v1, fs_v1, bs_v1,
             in_sem0, in_sem1, g_sem0, g_sem1, sc_sem0, sc_sem1):
    c = lax.axis_index("c")
    s = lax.axis_index("s")
    wid = s * NC + c

    r0 = pl.multiple_of(s * ROWS_PER_TILE, 8)
    pltpu.sync_copy(zero_hbm, acc_sh.at[pl.ds(r0, ROWS_PER_TILE)])
    plsc.subcore_barrier()

    lane0 = lax.iota(jnp.int32, 16)
    deg_col0 = jnp.maximum(1 - lane0, 0).astype(jnp.float32)

    def prefill(i, _):
        zf_v0[i, pl.ds(64, 16)] = deg_col0
        zb_v0[i, pl.ds(64, 16)] = deg_col0
        zf_v1[i, pl.ds(64, 16)] = deg_col0
        zb_v1[i, pl.ds(64, 16)] = deg_col0
        return ()

    lax.fori_loop(0, BLK, prefill, (), unroll=False)

    e0 = wid * EDGES_PER_W

    def fetch_in(blk, c_v, snd_v, rcv_v, sem):
        base = pl.multiple_of(e0 + blk * BLK, 8)
        pltpu.async_copy(c_hbm.at[pl.ds(base, BLK)], c_v, sem)
        pltpu.async_copy(snd_hbm.at[pl.ds(base, BLK)], snd_v, sem)
        pltpu.async_copy(rcv_hbm.at[pl.ds(base, BLK)], rcv_v, sem)

    def wait_in(c_v, snd_v, rcv_v, sem):
        pltpu.make_async_copy(c_hbm.at[pl.ds(0, BLK)], c_v, sem).wait()
        pltpu.make_async_copy(snd_hbm.at[pl.ds(0, BLK)], snd_v, sem).wait()
        pltpu.make_async_copy(rcv_hbm.at[pl.ds(0, BLK)], rcv_v, sem).wait()

    def fetch_g(snd_v, rcv_v, ys_v, yr_v, sem):
        pltpu.async_copy(y64_hbm.at[snd_v], ys_v, sem)
        pltpu.async_copy(y64_hbm.at[rcv_v], yr_v, sem)

    def wait_g(ys_v, yr_v, sem):
        pltpu.make_async_copy(y64_hbm.at[pl.ds(0, BLK)], ys_v, sem).wait()
        pltpu.make_async_copy(y64_hbm.at[pl.ds(0, BLK)], yr_v, sem).wait()

    def wait_sc(zf_v, zb_v, sem):
        pltpu.make_async_copy(out_hbm.at[pl.ds(0, BLK)], zf_v, sem).wait()
        pltpu.make_async_copy(out_hbm.at[pl.ds(0, BLK)], zb_v, sem).wait()

    def compute(c_v, ys_v, yr_v, zf_v, zb_v, snd_v, rcv_v, fs_v, bs_v):
        def edges(p4, _):
            for v in range(8):
                i = p4 * 8 + v
                cs = [c_v[i, pl.ds(16 * j, 16)] for j in range(4)]
                yss = [ys_v[i, pl.ds(16 * j, 16)] for j in range(4)]
                yrs = [yr_v[i, pl.ds(16 * j, 16)] for j in range(4)]
                zfs = [jnp.maximum(cs[j] + yss[j], 0.0) for j in range(4)]
                zbs = [jnp.maximum(cs[j] + yrs[j], 0.0) for j in range(4)]
                for j in range(4):
                    zf_v[i, pl.ds(16 * j, 16)] = zfs[j]
                    zb_v[i, pl.ds(16 * j, 16)] = zbs[j]
            return ()

        lax.fori_loop(0, BLK // 8, edges, (), unroll=False)
        for cc in range(BLK // 16):
            slc = pl.ds(16 * cc, 16)
            fs_v[slc] = rcv_v[slc]
            bs_v[slc] = snd_v[slc]

    def scatter(zf_v, zb_v, fs_v, bs_v, sem):
        pltpu.async_copy(zf_v, acc_sh.at[fs_v], sem, add=True)
        pltpu.async_copy(zb_v, acc_sh.at[bs_v], sem, add=True)

    fetch_in(0, c_v0, snd_v0, rcv_v0, in_sem0)
    fetch_in(1, c_v1, snd_v1, rcv_v1, in_sem1)
    wait_in(c_v0, snd_v0, rcv_v0, in_sem0)
    fetch_g(snd_v0, rcv_v0, ys_v0, yr_v0, g_sem0)

    def pair(q, _):
        a = 2 * q
        b = a + 1
        wait_g(ys_v0, yr_v0, g_sem0)
        wait_in(c_v1, snd_v1, rcv_v1, in_sem1)
        fetch_g(snd_v1, rcv_v1, ys_v1, yr_v1, g_sem1)

        @pl.when(q > 0)
        def _():
            wait_sc(zf_v0, zb_v0, sc_sem0)

        compute(c_v0, ys_v0, yr_v0, zf_v0, zb_v0, snd_v0, rcv_v0, fs_v0, bs_v0)
        scatter(zf_v0, zb_v0, fs_v0, bs_v0, sc_sem0)
        fetch_in(a + 2, c_v0, snd_v0, rcv_v0, in_sem0)
        wait_g(ys_v1, yr_v1, g_sem1)
        wait_in(c_v0, snd_v0, rcv_v0, in_sem0)
        fetch_g(snd_v0, rcv_v0, ys_v0, yr_v0, g_sem0)

        @pl.when(q > 0)
        def _():
            wait_sc(zf_v1, zb_v1, sc_sem1)

        compute(c_v1, ys_v1, yr_v1, zf_v1, zb_v1, snd_v1, rcv_v1, fs_v1, bs_v1)
        scatter(zf_v1, zb_v1, fs_v1, bs_v1, sc_sem1)

        @pl.when(q < N_PAIR - 1)
        def _():
            fetch_in(b + 2, c_v1, snd_v1, rcv_v1, in_sem1)

        return ()

    lax.fori_loop(0, N_PAIR, pair, (), unroll=False)

    wait_g(ys_v0, yr_v0, g_sem0)
    wait_sc(zf_v0, zb_v0, sc_sem0)
    compute(c_v0, ys_v0, yr_v0, zf_v0, zb_v0, snd_v0, rcv_v0, fs_v0, bs_v0)
    scatter(zf_v0, zb_v0, fs_v0, bs_v0, sc_sem0)
    wait_sc(zf_v0, zb_v0, sc_sem0)
    wait_sc(zf_v1, zb_v1, sc_sem1)

    plsc.subcore_barrier()
    out0 = pl.multiple_of(c * N_PAD + s * ROWS_PER_TILE, 8)
    pltpu.sync_copy(acc_sh.at[pl.ds(r0, ROWS_PER_TILE)],
                    out_hbm.at[pl.ds(out0, ROWS_PER_TILE)])


@functools.partial(jax.jit, static_argnames=())
def _sc_aggregate(c64, senders, receivers, y64):
    zero_blk = jnp.zeros((ROWS_PER_TILE, ACCW), jnp.float32)
    mesh = plsc.VectorSubcoreMesh(core_axis_name="c", subcore_axis_name="s",
                                  num_cores=NC, num_subcores=NS)
    buf = lambda: [
        pltpu.VMEM((BLK, 64), jnp.float32),
        pltpu.VMEM((BLK,), jnp.int32),
        pltpu.VMEM((BLK,), jnp.int32),
        pltpu.VMEM((BLK, 64), jnp.float32),
        pltpu.VMEM((BLK, 64), jnp.float32),
        pltpu.VMEM((BLK, ACCW), jnp.float32),
        pltpu.VMEM((BLK, ACCW), jnp.float32),
        pltpu.VMEM((BLK,), jnp.int32),
        pltpu.VMEM((BLK,), jnp.int32),
    ]
    kern = pl.kernel(
        _sc_body,
        out_type=jax.ShapeDtypeStruct((NC * N_PAD, ACCW), jnp.float32),
        mesh=mesh,
        compiler_params=pltpu.CompilerParams(use_tc_tiling_on_sc=False),
        scratch_types=(
            [pltpu.VMEM_SHARED((N_PAD, ACCW), jnp.float32)]
            + buf() + buf()
            + [pltpu.SemaphoreType.DMA] * 6
        ),
    )
    return kern(c64, senders.astype(jnp.int32),
                receivers.astype(jnp.int32), y64, zero_blk)


def _readout_body(acc_ref, x16_ref, ids_ref, wn_ref, win_ref, bn_ref,
                  wg_ref, bg_ref, wno_ref, bno_ref, node_ref, glob_ref):
    a0 = acc_ref[:N_PAD]
    a1 = acc_ref[N_PAD:]
    agg_sum = a0[:, :64] + a1[:, :64]
    deg = a0[:, 64:65] + a1[:, 64:65]
    agg = agg_sum / jnp.maximum(deg, 1.0)
    x16 = x16_ref[...]
    x = jnp.maximum(
        jnp.dot(x16, wn_ref[...], preferred_element_type=jnp.float32)
        + jnp.dot(agg, win_ref[...], preferred_element_type=jnp.float32)
        + bn_ref[...][None, :], 0.0)
    node_lin = jnp.dot(x, wno_ref[...], preferred_element_type=jnp.float32) \
        + bno_ref[...][None, :]
    node_ref[...] = 1.0 / (1.0 + jnp.exp(-node_lin))

    ids = ids_ref[...]
    onehot = (ids == lax.broadcasted_iota(jnp.int32, (N_PAD, N_GRAPHS), 1)
              ).astype(jnp.float32)
    g_sum = lax.dot_general(onehot, x, (((0,), (0,)), ((), ())),
                            preferred_element_type=jnp.float32)
    n_per = jnp.sum(onehot, axis=0)[:, None]
    g_mean = g_sum / jnp.maximum(n_per, 1.0)
    glob_lin = jnp.dot(g_mean, wg_ref[...], preferred_element_type=jnp.float32) \
        + bg_ref[...][None, :]
    glob_ref[...] = 1.0 / (1.0 + jnp.exp(-glob_lin))


def _readout(acc, x16, node_graph_ids, W_n, W_in, b_n, W_g, b_g, W_no, b_no):
    ids_p = jnp.full((N_PAD, 1), N_GRAPHS, jnp.int32).at[:N_NODES, 0].set(
        node_graph_ids.astype(jnp.int32))
    return pl.pallas_call(
        _readout_body,
        out_shape=(jax.ShapeDtypeStruct((N_PAD, 2), jnp.float32),
                   jax.ShapeDtypeStruct((N_GRAPHS, 2), jnp.float32)),
    )(acc, x16, ids_p, W_n, W_in, b_n, W_g, b_g, W_no, b_no)


def kernel(distances, edge_features, residues, node_features, senders,
           receivers, node_graph_ids, emb_table, We1, be1, We2, be2,
           Wn1, bn1, Wn2, bn2, W_e, W_s, b_e, W_n, W_in, b_n, W_g, b_g,
           W_no, b_no):
    c64 = _edge_encoder(distances, edge_features, We1, be1, We2, be2, W_e)
    x16, y64 = _node_encoder(residues, node_features, emb_table,
                             Wn1, bn1, Wn2, bn2, W_s, b_e)
    acc = _sc_aggregate(c64, senders, receivers, y64)
    node_out, global_out = _readout(acc, x16, node_graph_ids,
                                    W_n, W_in, b_n, W_g, b_g, W_no, b_no)
    return node_out[:N_NODES], global_out


# speedup vs baseline: 8.5184x; 1.4049x over previous
import functools

import numpy as np
import jax
import jax.numpy as jnp
from jax import lax
from jax.experimental import pallas as pl
from jax.experimental.pallas import tpu as pltpu
from jax.experimental.pallas import tpu_sc as plsc

N_NODES = 10000
N_PAD = 10240
N_EDGES = 320000
N_GRAPHS = 32
RBF_SIZE = 16
MIN_DIST = 0.0
MAX_DIST = 20.0

NC = 2
NS = 16
NW = NC * NS
EDGES_PER_W = N_EDGES // NW
BLK = 80
N_BLKS = EDGES_PER_W // BLK
N_PAIR = N_BLKS // 2
ACCW = 80
ROWS_PER_TILE = N_PAD // NS


EPACK = 8
EROWS = N_EDGES // EPACK


def _edge_body(d8_ref, ef8_ref, w1r_ref, w1e_ref, b1_ref, w2_ref,
               b2_ref, w3_ref, out_ref, c_scratch, dma_sem):
    ebp = d8_ref.shape[0]
    d8 = d8_ref[...]
    dx = jnp.concatenate(
        [jnp.broadcast_to(d8[:, g:g + 1], (ebp, RBF_SIZE))
         for g in range(EPACK)], axis=1)
    step = (MAX_DIST - MIN_DIST) / (RBF_SIZE - 1)
    centers = (lax.broadcasted_iota(jnp.int32, (1, EPACK * RBF_SIZE), 1)
               % RBF_SIZE).astype(jnp.float32) * step + MIN_DIST
    rbf = jnp.exp(-(dx - centers) ** 2)
    e4 = jnp.maximum(
        jnp.dot(rbf, w1r_ref[...], preferred_element_type=jnp.float32)
        + jnp.dot(ef8_ref[...], w1e_ref[...], preferred_element_type=jnp.float32)
        + b1_ref[...][None, :], 0.0)
    e8 = jnp.maximum(
        jnp.dot(e4, w2_ref[...], preferred_element_type=jnp.float32)
        + b2_ref[...][None, :], 0.0)
    c_scratch[...] = jnp.dot(e8, w3_ref[...], preferred_element_type=jnp.float32)
    i = pl.program_id(0)
    pltpu.sync_copy(c_scratch, out_ref.at[pl.ds(i * ebp, ebp)])


def _edge_encoder(distances, edge_features, We1, be1, We2, be2, W_e):
    d8 = distances.reshape(EROWS, EPACK)
    ef8 = edge_features.reshape(EROWS, EPACK * 16)
    w1r = jnp.zeros((128, 32), jnp.float32)
    w1e = jnp.zeros((128, 32), jnp.float32)
    w2 = jnp.zeros((32, 64), jnp.float32)
    w3 = jnp.zeros((64, 512), jnp.float32)
    for g in range(EPACK):
        w1r = w1r.at[16 * g:16 * g + 16, 4 * g:4 * g + 4].set(We1[:16])
        w1e = w1e.at[16 * g:16 * g + 16, 4 * g:4 * g + 4].set(We1[16:])
        w2 = w2.at[4 * g:4 * g + 4, 8 * g:8 * g + 8].set(We2)
        w3 = w3.at[8 * g:8 * g + 8, 64 * g:64 * g + 64].set(W_e)
    b1 = jnp.tile(be1, EPACK)
    b2 = jnp.tile(be2, EPACK)
    ebp = 4000
    grid = EROWS // ebp
    out = pl.pallas_call(
        _edge_body,
        grid=(grid,),
        in_specs=[
            pl.BlockSpec((ebp, EPACK), lambda i: (i, 0)),
            pl.BlockSpec((ebp, EPACK * 16), lambda i: (i, 0)),
            pl.BlockSpec((128, 32), lambda i: (0, 0)),
            pl.BlockSpec((128, 32), lambda i: (0, 0)),
            pl.BlockSpec((32,), lambda i: (0,)),
            pl.BlockSpec((32, 64), lambda i: (0, 0)),
            pl.BlockSpec((64,), lambda i: (0,)),
            pl.BlockSpec((64, 512), lambda i: (0, 0)),
        ],
        out_specs=pl.BlockSpec(memory_space=pltpu.HBM),
        out_shape=jax.ShapeDtypeStruct((EROWS, EPACK * 64), jnp.float32),
        scratch_shapes=[pltpu.VMEM((ebp, EPACK * 64), jnp.float32),
                        pltpu.SemaphoreType.DMA],
    )(d8, ef8, w1r, w1e, b1, w2, b2, w3)
    return out


def _node_body(res_ref, nf_ref, emb_ref, wn1_ref, bn1_ref, wn2_ref, bn2_ref,
               ws_ref, be_ref, x16_ref, y64_ref):
    res = res_ref[...]
    onehot = (res == lax.broadcasted_iota(jnp.int32, (N_PAD, 22), 1)
              ).astype(jnp.float32)
    emb = jnp.dot(onehot, emb_ref[...], preferred_element_type=jnp.float32)
    wn1 = wn1_ref[...]
    x8 = jnp.maximum(
        jnp.dot(emb, wn1[:32], preferred_element_type=jnp.float32)
        + jnp.dot(nf_ref[...], wn1[32:], preferred_element_type=jnp.float32)
        + bn1_ref[...][None, :], 0.0)
    x16 = jnp.maximum(
        jnp.dot(x8, wn2_ref[...], preferred_element_type=jnp.float32)
        + bn2_ref[...][None, :], 0.0)
    x16_ref[...] = x16
    y64_ref[...] = (jnp.dot(x16, ws_ref[...], preferred_element_type=jnp.float32)
                    + be_ref[...][None, :])


def _node_encoder(residues, node_features, emb_table, Wn1, bn1, Wn2, bn2, W_s, b_e):
    res_p = jnp.zeros((N_PAD, 1), jnp.int32).at[:N_NODES, 0].set(
        residues.astype(jnp.int32))
    nf_p = jnp.zeros((N_PAD, 96), jnp.float32).at[:N_NODES].set(node_features)
    return pl.pallas_call(
        _node_body,
        out_shape=(jax.ShapeDtypeStruct((N_PAD, 16), jnp.float32),
                   jax.ShapeDtypeStruct((N_PAD, 64), jnp.float32)),
    )(res_p, nf_p, emb_table, Wn1, bn1, Wn2, bn2, W_s, b_e)


def _sc_body(c_hbm, snd_hbm, rcv_hbm, y64_hbm, zero_hbm, out_hbm,
             acc_sh,
             c_v0, snd_v0, rcv_v0, ys_v0, yr_v0, zf_v0, zb_v0, fs_v0, bs_v0,
             c_v1, snd_v1, rcv_v1, ys_v1, yr_v1, zf_v1, zb_v1, fs_v1, bs_v1,
             in_sem0, in_sem1, g_sem0, g_sem1, sc_sem0, sc_sem1):
    c = lax.axis_index("c")
    s = lax.axis_index("s")
    wid = s * NC + c

    r0 = pl.multiple_of(s * ROWS_PER_TILE, 8)
    pltpu.sync_copy(zero_hbm, acc_sh.at[pl.ds(r0, ROWS_PER_TILE)])
    plsc.subcore_barrier()

    lane0 = lax.iota(jnp.int32, 16)
    deg_col0 = jnp.maximum(1 - lane0, 0).astype(jnp.float32)

    def prefill(i, _):
        zf_v0[i, pl.ds(64, 16)] = deg_col0
        zb_v0[i, pl.ds(64, 16)] = deg_col0
        zf_v1[i, pl.ds(64, 16)] = deg_col0
        zb_v1[i, pl.ds(64, 16)] = deg_col0
        return ()

    lax.fori_loop(0, BLK, prefill, (), unroll=False)

    e0 = wid * EDGES_PER_W

    def fetch_in(blk, c_v, snd_v, rcv_v, sem):
        base = pl.multiple_of(e0 + blk * BLK, 8)
        pltpu.async_copy(c_hbm.at[pl.ds(base // EPACK, BLK // EPACK)], c_v, sem)
        pltpu.async_copy(snd_hbm.at[pl.ds(base, BLK)], snd_v, sem)
        pltpu.async_copy(rcv_hbm.at[pl.ds(base, BLK)], rcv_v, sem)

    def wait_in(c_v, snd_v, rcv_v, sem):
        pltpu.make_async_copy(c_hbm.at[pl.ds(0, BLK // EPACK)], c_v, sem).wait()
        pltpu.make_async_copy(snd_hbm.at[pl.ds(0, BLK)], snd_v, sem).wait()
        pltpu.make_async_copy(rcv_hbm.at[pl.ds(0, BLK)], rcv_v, sem).wait()

    def fetch_g(snd_v, rcv_v, ys_v, yr_v, sem):
        pltpu.async_copy(y64_hbm.at[snd_v], ys_v, sem)
        pltpu.async_copy(y64_hbm.at[rcv_v], yr_v, sem)

    def wait_g(ys_v, yr_v, sem):
        pltpu.make_async_copy(y64_hbm.at[pl.ds(0, BLK)], ys_v, sem).wait()
        pltpu.make_async_copy(y64_hbm.at[pl.ds(0, BLK)], yr_v, sem).wait()

    def wait_sc(zf_v, zb_v, sem):
        pltpu.make_async_copy(out_hbm.at[pl.ds(0, BLK)], zf_v, sem).wait()
        pltpu.make_async_copy(out_hbm.at[pl.ds(0, BLK)], zb_v, sem).wait()

    def compute(c_v, ys_v, yr_v, zf_v, zb_v, snd_v, rcv_v, fs_v, bs_v):
        def edges(p4, _):
            for v in range(8):
                i = p4 * 8 + v
                cs = [c_v[i // EPACK, pl.ds((i % EPACK) * 64 + 16 * j, 16)]
                      for j in range(4)]
                yss = [ys_v[i, pl.ds(16 * j, 16)] for j in range(4)]
                yrs = [yr_v[i, pl.ds(16 * j, 16)] for j in range(4)]
                zfs = [jnp.maximum(cs[j] + yss[j], 0.0) for j in range(4)]
                zbs = [jnp.maximum(cs[j] + yrs[j], 0.0) for j in range(4)]
                for j in range(4):
                    zf_v[i, pl.ds(16 * j, 16)] = zfs[j]
                    zb_v[i, pl.ds(16 * j, 16)] = zbs[j]
            return ()

        lax.fori_loop(0, BLK // 8, edges, (), unroll=False)
        for cc in range(BLK // 16):
            slc = pl.ds(16 * cc, 16)
            fs_v[slc] = rcv_v[slc]
            bs_v[slc] = snd_v[slc]

    def scatter(zf_v, zb_v, fs_v, bs_v, sem):
        pltpu.async_copy(zf_v, acc_sh.at[fs_v], sem, add=True)
        pltpu.async_copy(zb_v, acc_sh.at[bs_v], sem, add=True)

    fetch_in(0, c_v0, snd_v0, rcv_v0, in_sem0)
    fetch_in(1, c_v1, snd_v1, rcv_v1, in_sem1)
    wait_in(c_v0, snd_v0, rcv_v0, in_sem0)
    fetch_g(snd_v0, rcv_v0, ys_v0, yr_v0, g_sem0)

    def pair(q, _):
        a = 2 * q
        b = a + 1
        wait_g(ys_v0, yr_v0, g_sem0)
        wait_in(c_v1, snd_v1, rcv_v1, in_sem1)
        fetch_g(snd_v1, rcv_v1, ys_v1, yr_v1, g_sem1)

        @pl.when(q > 0)
        def _():
            wait_sc(zf_v0, zb_v0, sc_sem0)

        compute(c_v0, ys_v0, yr_v0, zf_v0, zb_v0, snd_v0, rcv_v0, fs_v0, bs_v0)
        scatter(zf_v0, zb_v0, fs_v0, bs_v0, sc_sem0)
        fetch_in(a + 2, c_v0, snd_v0, rcv_v0, in_sem0)
        wait_g(ys_v1, yr_v1, g_sem1)
        wait_in(c_v0, snd_v0, rcv_v0, in_sem0)
        fetch_g(snd_v0, rcv_v0, ys_v0, yr_v0, g_sem0)

        @pl.when(q > 0)
        def _():
            wait_sc(zf_v1, zb_v1, sc_sem1)

        compute(c_v1, ys_v1, yr_v1, zf_v1, zb_v1, snd_v1, rcv_v1, fs_v1, bs_v1)
        scatter(zf_v1, zb_v1, fs_v1, bs_v1, sc_sem1)

        @pl.when(q < N_PAIR - 1)
        def _():
            fetch_in(b + 2, c_v1, snd_v1, rcv_v1, in_sem1)

        return ()

    lax.fori_loop(0, N_PAIR, pair, (), unroll=False)

    wait_g(ys_v0, yr_v0, g_sem0)
    wait_sc(zf_v0, zb_v0, sc_sem0)
    compute(c_v0, ys_v0, yr_v0, zf_v0, zb_v0, snd_v0, rcv_v0, fs_v0, bs_v0)
    scatter(zf_v0, zb_v0, fs_v0, bs_v0, sc_sem0)
    wait_sc(zf_v0, zb_v0, sc_sem0)
    wait_sc(zf_v1, zb_v1, sc_sem1)

    plsc.subcore_barrier()
    out0 = pl.multiple_of(c * N_PAD + s * ROWS_PER_TILE, 8)
    pltpu.sync_copy(acc_sh.at[pl.ds(r0, ROWS_PER_TILE)],
                    out_hbm.at[pl.ds(out0, ROWS_PER_TILE)])


@functools.partial(jax.jit, static_argnames=())
def _sc_aggregate(c64, senders, receivers, y64):
    zero_blk = jnp.zeros((ROWS_PER_TILE, ACCW), jnp.float32)
    mesh = plsc.VectorSubcoreMesh(core_axis_name="c", subcore_axis_name="s",
                                  num_cores=NC, num_subcores=NS)
    buf = lambda: [
        pltpu.VMEM((BLK // EPACK, EPACK * 64), jnp.float32),
        pltpu.VMEM((BLK,), jnp.int32),
        pltpu.VMEM((BLK,), jnp.int32),
        pltpu.VMEM((BLK, 64), jnp.float32),
        pltpu.VMEM((BLK, 64), jnp.float32),
        pltpu.VMEM((BLK, ACCW), jnp.float32),
        pltpu.VMEM((BLK, ACCW), jnp.float32),
        pltpu.VMEM((BLK,), jnp.int32),
        pltpu.VMEM((BLK,), jnp.int32),
    ]
    kern = pl.kernel(
        _sc_body,
        out_type=jax.ShapeDtypeStruct((NC * N_PAD, ACCW), jnp.float32),
        mesh=mesh,
        compiler_params=pltpu.CompilerParams(use_tc_tiling_on_sc=False),
        scratch_types=(
            [pltpu.VMEM_SHARED((N_PAD, ACCW), jnp.float32)]
            + buf() + buf()
            + [pltpu.SemaphoreType.DMA] * 6
        ),
    )
    return kern(c64, senders.astype(jnp.int32),
                receivers.astype(jnp.int32), y64, zero_blk)


def _readout_body(acc_ref, x16_ref, ids_ref, wn_ref, win_ref, bn_ref,
                  wg_ref, bg_ref, wno_ref, bno_ref, node_ref, glob_ref):
    a0 = acc_ref[:N_PAD]
    a1 = acc_ref[N_PAD:]
    agg_sum = a0[:, :64] + a1[:, :64]
    deg = a0[:, 64:65] + a1[:, 64:65]
    agg = agg_sum / jnp.maximum(deg, 1.0)
    x16 = x16_ref[...]
    x = jnp.maximum(
        jnp.dot(x16, wn_ref[...], preferred_element_type=jnp.float32)
        + jnp.dot(agg, win_ref[...], preferred_element_type=jnp.float32)
        + bn_ref[...][None, :], 0.0)
    node_lin = jnp.dot(x, wno_ref[...], preferred_element_type=jnp.float32) \
        + bno_ref[...][None, :]
    node_ref[...] = 1.0 / (1.0 + jnp.exp(-node_lin))

    ids = ids_ref[...]
    onehot = (ids == lax.broadcasted_iota(jnp.int32, (N_PAD, N_GRAPHS), 1)
              ).astype(jnp.float32)
    g_sum = lax.dot_general(onehot, x, (((0,), (0,)), ((), ())),
                            preferred_element_type=jnp.float32)
    n_per = jnp.sum(onehot, axis=0)[:, None]
    g_mean = g_sum / jnp.maximum(n_per, 1.0)
    glob_lin = jnp.dot(g_mean, wg_ref[...], preferred_element_type=jnp.float32) \
        + bg_ref[...][None, :]
    glob_ref[...] = 1.0 / (1.0 + jnp.exp(-glob_lin))


def _readout(acc, x16, node_graph_ids, W_n, W_in, b_n, W_g, b_g, W_no, b_no):
    ids_p = jnp.full((N_PAD, 1), N_GRAPHS, jnp.int32).at[:N_NODES, 0].set(
        node_graph_ids.astype(jnp.int32))
    return pl.pallas_call(
        _readout_body,
        out_shape=(jax.ShapeDtypeStruct((N_PAD, 2), jnp.float32),
                   jax.ShapeDtypeStruct((N_GRAPHS, 2), jnp.float32)),
    )(acc, x16, ids_p, W_n, W_in, b_n, W_g, b_g, W_no, b_no)


def kernel(distances, edge_features, residues, node_features, senders,
           receivers, node_graph_ids, emb_table, We1, be1, We2, be2,
           Wn1, bn1, Wn2, bn2, W_e, W_s, b_e, W_n, W_in, b_n, W_g, b_g,
           W_no, b_no):
    c64 = _edge_encoder(distances, edge_features, We1, be1, We2, be2, W_e)
    x16, y64 = _node_encoder(residues, node_features, emb_table,
                             Wn1, bn1, Wn2, bn2, W_s, b_e)
    acc = _sc_aggregate(c64, senders, receivers, y64)
    node_out, global_out = _readout(acc, x16, node_graph_ids,
                                    W_n, W_in, b_n, W_g, b_g, W_no, b_no)
    return node_out[:N_NODES], global_out
